# baseline (speedup 1.0000x reference)
import functools

import jax
import jax.numpy as jnp
from jax import lax
from jax.experimental import pallas as pl
from jax.experimental.pallas import tpu as pltpu
from jax.experimental.pallas import tpu_sc as plsc

B = 4
S = 8192
D = 1024
NC = 2
NS = 16
NW = NC * NS
ROWS_PER_W = S // NW
CHUNK_ROWS = 32
CHUNK = CHUNK_ROWS * D
N_CHUNKS = ROWS_PER_W // CHUNK_ROWS


def _sc_body(x_hbm, tab_hbm, out_hbm, xv, tabv):
    wid = lax.axis_index("s") * NC + lax.axis_index("c")
    seq_base = wid * (ROWS_PER_W * D)
    for t in range(N_CHUNKS):
        toff = seq_base + t * CHUNK
        pltpu.sync_copy(tab_hbm.at[pl.ds(toff, CHUNK)], tabv)
        for b in range(B):
            xoff = b * (S * D) + toff
            pltpu.sync_copy(x_hbm.at[pl.ds(xoff, CHUNK)], xv)

            @plsc.parallel_loop(0, CHUNK, 16, unroll=8)
            def _add(off):
                xv[pl.ds(off, 16)] = xv[pl.ds(off, 16)] + tabv[pl.ds(off, 16)]

            pltpu.sync_copy(xv, out_hbm.at[pl.ds(xoff, CHUNK)])


def kernel(x, position_table):
    batch, seq_len, d_model = x.shape
    mesh = plsc.VectorSubcoreMesh(core_axis_name="c", subcore_axis_name="s")
    run = functools.partial(
        pl.kernel,
        out_type=jax.ShapeDtypeStruct((batch * seq_len * d_model,), jnp.float32),
        mesh=mesh,
        scratch_types=[
            pltpu.VMEM((CHUNK,), jnp.float32),
            pltpu.VMEM((CHUNK,), jnp.float32),
        ],
    )(_sc_body)
    out = run(x.reshape(-1), position_table[:seq_len].reshape(-1))
    return out.reshape(x.shape)


# speedup vs baseline: 4.9884x; 4.9884x over previous
import jax
import jax.numpy as jnp
from jax.experimental import pallas as pl
from jax.experimental.pallas import tpu as pltpu

BLOCK_S = 1024


def _add_body(x_ref, tab_ref, out_ref):
    out_ref[0, :, :] = x_ref[0, :, :] + tab_ref[:, :]


def kernel(x, position_table):
    batch, seq_len, d_model = x.shape
    table = position_table[:seq_len]
    grid = (seq_len // BLOCK_S, batch)
    return pl.pallas_call(
        _add_body,
        grid=grid,
        in_specs=[
            pl.BlockSpec((1, BLOCK_S, d_model), lambda s, b: (b, s, 0)),
            pl.BlockSpec((BLOCK_S, d_model), lambda s, b: (s, 0)),
        ],
        out_specs=pl.BlockSpec((1, BLOCK_S, d_model), lambda s, b: (b, s, 0)),
        out_shape=jax.ShapeDtypeStruct(x.shape, x.dtype),
        compiler_params=pltpu.CompilerParams(
            dimension_semantics=("arbitrary", "arbitrary"),
        ),
    )(x, table)


# speedup vs baseline: 5.2019x; 1.0428x over previous
import jax
import jax.numpy as jnp
from jax.experimental import pallas as pl
from jax.experimental.pallas import tpu as pltpu

BLOCK_S = 2048


def _add_body(x_ref, tab_ref, out_ref):
    out_ref[0, :, :] = x_ref[0, :, :] + tab_ref[:, :]


def kernel(x, position_table):
    batch, seq_len, d_model = x.shape
    table = position_table[:seq_len]
    grid = (seq_len // BLOCK_S, batch)
    return pl.pallas_call(
        _add_body,
        grid=grid,
        in_specs=[
            pl.BlockSpec((1, BLOCK_S, d_model), lambda s, b: (b, s, 0)),
            pl.BlockSpec((BLOCK_S, d_model), lambda s, b: (s, 0)),
        ],
        out_specs=pl.BlockSpec((1, BLOCK_S, d_model), lambda s, b: (b, s, 0)),
        out_shape=jax.ShapeDtypeStruct(x.shape, x.dtype),
        compiler_params=pltpu.CompilerParams(
            dimension_semantics=("arbitrary", "arbitrary"),
        ),
    )(x, table)
